# baseline (speedup 1.0000x reference)
import jax
import jax.numpy as jnp
from jax import lax
from jax.experimental import pallas as pl
from jax.experimental.pallas import tpu as pltpu
from jax.experimental.pallas import tpu_sc as plsc

NC = 2
NS = 16
NW = NC * NS

BATCH = 4096
SEQ = 200
D_MODEL = 64
NPAIR = 500000
BBLK = BATCH // NW
NBUF = 4
N_GROUPS = SEQ // NBUF


def _gather_body(idxT_hbm, table2_hbm, outT_hbm, idx_v, pidx_v, rows_v,
                 trans_v, gsems, ssems):
    t = lax.axis_index("s") * NC + lax.axis_index("c")
    b0 = t * BBLK

    pltpu.sync_copy(idxT_hbm.at[:, pl.ds(b0, BBLK)], idx_v)

    iota = lax.iota(jnp.int32, 16)

    def compute_pidx(s, b):
        for k in range(BBLK // 16):
            v = idx_v[s, pl.ds(16 * k, 16)]
            pidx_v[b, pl.ds(16 * k, 16)] = lax.shift_right_logical(v, 1)

    def gather_copy(b):
        return pltpu.make_async_copy(
            table2_hbm.at[pidx_v.at[b]], rows_v.at[b], gsems.at[b])

    def store_copy(s, b):
        dst = outT_hbm.at[s].at[:, pl.ds(b0, BBLK)]
        return pltpu.make_async_copy(trans_v.at[b], dst, ssems.at[b])

    def transpose(s, b):
        voffs = tuple(
            lax.shift_left(
                lax.bitwise_and(idx_v[s, pl.ds(16 * rb, 16)], 1), 6)
            for rb in range(BBLK // 16))

        def fchunk(c, offs):
            f0 = c * 8
            for df in range(8):
                f = f0 + df
                for rb in range(BBLK // 16):
                    vals = plsc.load_gather(
                        rows_v.at[b], [iota + 16 * rb, offs[rb] + f])
                    trans_v[b, f, pl.ds(16 * rb, 16)] = vals
            return offs

        lax.fori_loop(0, D_MODEL // 8, fchunk, voffs)

    for b in range(NBUF):
        compute_pidx(b, b)
        gather_copy(b).start()

    for b in range(NBUF):
        s = b
        gather_copy(b).wait()
        transpose(s, b)
        store_copy(s, b).start()
        compute_pidx(s + NBUF, b)
        gather_copy(b).start()

    def group(g, _):
        for b in range(NBUF):
            s = g * NBUF + b
            gather_copy(b).wait()
            store_copy(s - NBUF, b).wait()
            transpose(s, b)
            store_copy(s, b).start()
            compute_pidx(s + NBUF, b)
            gather_copy(b).start()
        return ()

    lax.fori_loop(1, N_GROUPS - 1, group, ())

    for b in range(NBUF):
        s = (N_GROUPS - 1) * NBUF + b
        gather_copy(b).wait()
        store_copy(s - NBUF, b).wait()
        transpose(s, b)
        store_copy(s, b).start()
    for b in range(NBUF):
        s = (N_GROUPS - 1) * NBUF + b
        store_copy(s, b).wait()


@jax.jit
def _embed(indices_t, table2):
    mesh = plsc.VectorSubcoreMesh(core_axis_name="c", subcore_axis_name="s")
    f = pl.kernel(
        _gather_body,
        out_type=jax.ShapeDtypeStruct((SEQ, D_MODEL, BATCH), jnp.float32),
        mesh=mesh,
        scratch_types=[
            pltpu.VMEM((SEQ, BBLK), jnp.int32),
            pltpu.VMEM((NBUF, BBLK), jnp.int32),
            pltpu.VMEM((NBUF, BBLK, 128), jnp.float32),
            pltpu.VMEM((NBUF, D_MODEL, BBLK), jnp.float32),
            pltpu.SemaphoreType.DMA((NBUF,)),
            pltpu.SemaphoreType.DMA((NBUF,)),
        ],
        compiler_params=pltpu.CompilerParams(needs_layout_passes=False),
    )
    return f(indices_t, table2)


def kernel(indices, table):
    idx_t = indices.T.astype(jnp.int32)
    table2 = table.reshape(NPAIR, 128)
    out_t = _embed(idx_t, table2)
    return jnp.transpose(out_t, (2, 0, 1))


# speedup vs baseline: 1.4625x; 1.4625x over previous
import jax
import jax.numpy as jnp
from jax import lax
from jax.experimental import pallas as pl
from jax.experimental.pallas import tpu as pltpu
from jax.experimental.pallas import tpu_sc as plsc

NC = 2
NS = 16
NW = NC * NS

BATCH = 4096
SEQ = 200
D_MODEL = 64
NPAIR = 500000
BBLK = BATCH // NW
NBUF = 4
N_GROUPS = SEQ // NBUF


def _gather_body(idxT_hbm, table2_hbm, outT_hbm, idx_v, pidx_v, rows_v,
                 trans_v, gsems, ssems):
    t = lax.axis_index("s") * NC + lax.axis_index("c")
    b0 = t * BBLK

    pltpu.sync_copy(idxT_hbm.at[:, pl.ds(b0, BBLK)], idx_v)

    iota = lax.iota(jnp.int32, 16)

    def compute_pidx(s, b):
        for k in range(BBLK // 16):
            v = idx_v[s, pl.ds(16 * k, 16)]
            pidx_v[b, pl.ds(16 * k, 16)] = lax.shift_right_logical(v, 1)

    def gather_copy(b):
        return pltpu.make_async_copy(
            table2_hbm.at[pidx_v.at[b]], rows_v.at[b], gsems.at[b])

    def store_copy(s, b):
        dst = outT_hbm.at[s].at[:, pl.ds(b0, BBLK)]
        return pltpu.make_async_copy(trans_v.at[b], dst, ssems.at[b])

    def transpose(s, b):
        voffs = tuple(
            lax.shift_left(
                lax.bitwise_and(idx_v[s, pl.ds(16 * rb, 16)], 1), 6)
            for rb in range(BBLK // 16))

        @plsc.parallel_loop(0, D_MODEL, 1, unroll=8, carry=voffs)
        def frow(f, offs):
            for rb in range(BBLK // 16):
                vals = plsc.load_gather(
                    rows_v.at[b], [iota + 16 * rb, offs[rb] + f])
                trans_v[b, f, pl.ds(16 * rb, 16)] = vals
            return offs

    for b in range(NBUF):
        compute_pidx(b, b)
        gather_copy(b).start()

    for b in range(NBUF):
        s = b
        gather_copy(b).wait()
        transpose(s, b)
        store_copy(s, b).start()
        compute_pidx(s + NBUF, b)
        gather_copy(b).start()

    def group(g, _):
        for b in range(NBUF):
            s = g * NBUF + b
            gather_copy(b).wait()
            store_copy(s - NBUF, b).wait()
            transpose(s, b)
            store_copy(s, b).start()
            compute_pidx(s + NBUF, b)
            gather_copy(b).start()
        return ()

    lax.fori_loop(1, N_GROUPS - 1, group, ())

    for b in range(NBUF):
        s = (N_GROUPS - 1) * NBUF + b
        gather_copy(b).wait()
        store_copy(s - NBUF, b).wait()
        transpose(s, b)
        store_copy(s, b).start()
    for b in range(NBUF):
        s = (N_GROUPS - 1) * NBUF + b
        store_copy(s, b).wait()


@jax.jit
def _embed(indices_t, table2):
    mesh = plsc.VectorSubcoreMesh(core_axis_name="c", subcore_axis_name="s")
    f = pl.kernel(
        _gather_body,
        out_type=jax.ShapeDtypeStruct((SEQ, D_MODEL, BATCH), jnp.float32),
        mesh=mesh,
        scratch_types=[
            pltpu.VMEM((SEQ, BBLK), jnp.int32),
            pltpu.VMEM((NBUF, BBLK), jnp.int32),
            pltpu.VMEM((NBUF, BBLK, 128), jnp.float32),
            pltpu.VMEM((NBUF, D_MODEL, BBLK), jnp.float32),
            pltpu.SemaphoreType.DMA((NBUF,)),
            pltpu.SemaphoreType.DMA((NBUF,)),
        ],
        compiler_params=pltpu.CompilerParams(needs_layout_passes=False),
    )
    return f(indices_t, table2)


def kernel(indices, table):
    idx_t = indices.T.astype(jnp.int32)
    table2 = table.reshape(NPAIR, 128)
    out_t = _embed(idx_t, table2)
    return jnp.transpose(out_t, (2, 0, 1))
